# baseline (speedup 1.0000x reference)
import functools

import jax
import jax.numpy as jnp
from jax import lax
from jax.experimental import pallas as pl
from jax.experimental.pallas import tpu as pltpu
from jax.experimental.pallas import tpu_sc as plsc

_T = 2048
_D = 1024
_F = 2048
_E = 8
_K = 2
_J = _K * _T
_B = 256
_G = 23
_PAD = _G * _B

_NC = 2
_NS = 16
_NW = _NC * _NS
_JC = _J // _NW
_PC = _PAD // _NW
_TT = _T // _NW


def _route_body(hs_ref, gwt_ref, scores_ref, counts_ref, cum_ref,
                pos_ref, ppad_ref, bexp_ref, s0x_ref, s1x_ref, hsb_ref):
    hs = hs_ref[...]
    logits = lax.dot_general(
        hs.astype(jnp.bfloat16), gwt_ref[...].astype(jnp.bfloat16),
        (((1,), (0,)), ((), ())), preferred_element_type=jnp.float32)
    m = jnp.max(logits, axis=1, keepdims=True)
    ex = jnp.exp(logits - m)
    probs = ex / jnp.sum(ex, axis=1, keepdims=True)
    lane = lax.broadcasted_iota(jnp.int32, (_T, _E), 1)
    m1 = jnp.max(probs, axis=1, keepdims=True)
    i1 = jnp.min(jnp.where(probs == m1, lane, _E), axis=1, keepdims=True)
    masked = jnp.where(lane == i1, -jnp.inf, probs)
    m2 = jnp.max(masked, axis=1, keepdims=True)
    i2 = jnp.min(jnp.where(masked == m2, lane, _E), axis=1, keepdims=True)
    scores_ref[...] = jnp.concatenate([m1, m2], axis=1)
    s0x_ref[...] = jnp.broadcast_to(m1, (_T, 16))
    s1x_ref[...] = jnp.broadcast_to(m2, (_T, 16))
    hsb_ref[...] = hs.astype(jnp.bfloat16)

    oh = jnp.concatenate([(lane == i1).astype(jnp.int32),
                          (lane == i2).astype(jnp.int32)], axis=0)
    counts = jnp.sum(oh, axis=0, keepdims=True)
    counts_ref[...] = counts

    def lane_cumsum(v):
        for d in (1, 2, 4):
            v = v + jnp.concatenate(
                [jnp.zeros((1, d), v.dtype), v[:, :_E - d]], axis=1)
        return v

    cum = lane_cumsum(counts)
    cum_ref[...] = cum
    start = cum - counts
    pb = (counts + (_B - 1)) // _B
    startb = lane_cumsum(pb) - pb
    startp = startb * _B

    inc = oh
    d = 1
    while d < _J:
        inc = inc + jnp.concatenate(
            [jnp.zeros((d, _E), jnp.int32), inc[:_J - d, :]], axis=0)
        d *= 2
    rank = jnp.sum(oh * inc, axis=1, keepdims=True) - 1
    pos_ref[...] = rank + jnp.sum(oh * start, axis=1, keepdims=True)
    ppad_ref[...] = rank + jnp.sum(oh * startp, axis=1, keepdims=True)

    eye = (lax.broadcasted_iota(jnp.int32, (_E, _E), 0)
           == lax.broadcasted_iota(jnp.int32, (_E, _E), 1)).astype(jnp.float32)
    startb_sub = lax.dot_general(
        eye, startb.astype(jnp.float32), (((1,), (1,)), ((), ())),
        preferred_element_type=jnp.float32)
    biota = lax.broadcasted_iota(jnp.int32, (_E, _G), 1).astype(jnp.float32)
    cmp = (startb_sub <= biota).astype(jnp.int32)
    bexp_ref[...] = jnp.sum(cmp, axis=0, keepdims=True) - 1


def _route(hidden, gwt):
    return pl.pallas_call(
        _route_body,
        out_shape=(
            jax.ShapeDtypeStruct((_T, _K), jnp.float32),
            jax.ShapeDtypeStruct((1, _E), jnp.int32),
            jax.ShapeDtypeStruct((1, _E), jnp.int32),
            jax.ShapeDtypeStruct((_J, 1), jnp.int32),
            jax.ShapeDtypeStruct((_J, 1), jnp.int32),
            jax.ShapeDtypeStruct((1, _G), jnp.int32),
            jax.ShapeDtypeStruct((_T, 16), jnp.float32),
            jax.ShapeDtypeStruct((_T, 16), jnp.float32),
            jax.ShapeDtypeStruct((_T, _D), jnp.bfloat16),
        ),
    )(hidden, gwt)


def _tables_body(pos_hbm, ppad_hbm, ms_hbm, sp_hbm, posv, ppv, msbuf, slots):
    wid = lax.axis_index("s") * _NC + lax.axis_index("c")
    mlo = wid * _JC
    slo = wid * _PC
    pltpu.sync_copy(pos_hbm, posv)
    pltpu.sync_copy(ppad_hbm, ppv)
    for i in range(_PC // 16 + 1):
        slots[pl.ds(i * 16, 16)] = jnp.zeros((16,), jnp.int32)

    @plsc.parallel_loop(0, _J // 16, unroll=8)
    def chunk(c):
        pv = posv[pl.ds(c * 16, 16)]
        pp = ppv[pl.ds(c * 16, 16)]
        tok = (c * 16 + lax.iota(jnp.int32, 16)) & (_T - 1)
        plsc.store_scatter(msbuf, [pv - mlo], tok,
                           mask=(pv >= mlo) & (pv < mlo + _JC))
        plsc.store_scatter(slots, [pp - slo], tok,
                           mask=(pp >= slo) & (pp < slo + _PC))

    pltpu.sync_copy(msbuf, ms_hbm.at[pl.ds(mlo, _JC)])
    pltpu.sync_copy(slots.at[pl.ds(0, _PC)], sp_hbm.at[pl.ds(slo, _PC)])


def _gemm_body(be_ref, nb_ref, slots_ref, hid_ref, w1_ref, w2_ref, y_ref):
    i = pl.program_id(0)

    @pl.when(i < nb_ref[0])
    def _():
        s_lane = slots_ref[0].astype(jnp.float32)
        eye = (lax.broadcasted_iota(jnp.int32, (_B, _B), 0)
               == lax.broadcasted_iota(jnp.int32, (_B, _B), 1)
               ).astype(jnp.float32)
        s_col = lax.dot_general(eye, s_lane, (((1,), (1,)), ((), ())),
                                preferred_element_type=jnp.float32)
        t_iota = lax.broadcasted_iota(jnp.int32, (_B, _T), 1).astype(jnp.float32)
        p_blk = (s_col == t_iota).astype(jnp.bfloat16)
        xb = lax.dot_general(p_blk, hid_ref[...], (((1,), (0,)), ((), ())),
                             preferred_element_type=jnp.float32)
        h = lax.dot_general(xb, w1_ref[0],
                            (((1,), (0,)), ((), ())),
                            preferred_element_type=jnp.float32)
        h = h * (1.0 / (1.0 + jnp.exp(-h)))
        y_ref[...] = lax.dot_general(h, w2_ref[0],
                                     (((1,), (0,)), ((), ())),
                                     preferred_element_type=jnp.float32)


def _gemm(bexp, nblk, slots_r, hid_bf, w_in, w_out):
    def _clamped(i, be, nb):
        return jnp.minimum(i, nb[0] - 1)

    grid_spec = pltpu.PrefetchScalarGridSpec(
        num_scalar_prefetch=2,
        grid=(_G,),
        in_specs=[
            pl.BlockSpec((1, 1, _B), lambda i, be, nb: (i, 0, 0)),
            pl.BlockSpec((_T, _D), lambda i, be, nb: (0, 0)),
            pl.BlockSpec((1, _D, _F),
                         lambda i, be, nb: (be[_clamped(i, be, nb)], 0, 0)),
            pl.BlockSpec((1, _F, _D),
                         lambda i, be, nb: (be[_clamped(i, be, nb)], 0, 0)),
        ],
        out_specs=pl.BlockSpec((_B, _D), lambda i, be, nb: (i, 0)),
    )
    return pl.pallas_call(
        _gemm_body,
        grid_spec=grid_spec,
        out_shape=jax.ShapeDtypeStruct((_PAD, _D), jnp.float32),
    )(bexp, nblk, slots_r, hid_bf, w_in, w_out)


def _combine_body(yp_hbm, ppad_hbm, s0_hbm, s1_hbm, out_hbm,
                  pp0, pp1, s0b, s1b, y0, y1, sem):
    wid = lax.axis_index("s") * _NC + lax.axis_index("c")
    tbase = wid * _TT
    for c in range(_TT // 32):
        t0 = tbase + c * 32
        pltpu.sync_copy(ppad_hbm.at[pl.ds(t0, 32)], pp0)
        pltpu.sync_copy(ppad_hbm.at[pl.ds(_T + t0, 32)], pp1)
        pltpu.sync_copy(s0_hbm.at[pl.ds(t0, 32)], s0b)
        pltpu.sync_copy(s1_hbm.at[pl.ds(t0, 32)], s1b)
        pltpu.async_copy(yp_hbm.at[pp0], y0, sem).wait()
        pltpu.async_copy(yp_hbm.at[pp1], y1, sem).wait()

        def row(r, _):
            s0v = s0b[r, :]
            s1v = s1b[r, :]

            def col(cc, _):
                sl = pl.ds(cc * 16, 16)
                y0[r, sl] = s0v * y0[r, sl] + s1v * y1[r, sl]
                return 0

            return lax.fori_loop(0, _D // 16, col, 0)

        lax.fori_loop(0, 32, row, 0)
        pltpu.sync_copy(y0, out_hbm.at[pl.ds(t0, 32)])


@functools.lru_cache(maxsize=1)
def _sc_kernels():
    mesh = plsc.VectorSubcoreMesh(
        core_axis_name="c", subcore_axis_name="s",
        num_cores=_NC, num_subcores=_NS)
    tables_k = pl.kernel(
        _tables_body,
        out_type=(jax.ShapeDtypeStruct((_J,), jnp.int32),
                  jax.ShapeDtypeStruct((_PAD,), jnp.int32)),
        mesh=mesh,
        compiler_params=pltpu.CompilerParams(needs_layout_passes=False),
        scratch_types=[pltpu.VMEM((_J,), jnp.int32),
                       pltpu.VMEM((_J,), jnp.int32),
                       pltpu.VMEM((_JC,), jnp.int32),
                       pltpu.VMEM((192,), jnp.int32)])
    combine_k = pl.kernel(
        _combine_body,
        out_type=jax.ShapeDtypeStruct((_T, _D), jnp.float32),
        mesh=mesh,
        scratch_types=[pltpu.VMEM((32,), jnp.int32),
                       pltpu.VMEM((32,), jnp.int32),
                       pltpu.VMEM((32, 16), jnp.float32),
                       pltpu.VMEM((32, 16), jnp.float32),
                       pltpu.VMEM((32, _D), jnp.float32),
                       pltpu.VMEM((32, _D), jnp.float32),
                       pltpu.SemaphoreType.DMA])
    return tables_k, combine_k


def kernel(hidden_states, gate_w, w_in, w_out):
    (scores, counts2, cum2, pos2, ppad2, bexp2,
     s0x, s1x, hs_bf) = _route(hidden_states, gate_w.T)
    counts = counts2.reshape(_E)
    pos = pos2.reshape(_J)
    ppad = ppad2.reshape(_J)
    bexp = bexp2.reshape(_G)
    nblk = jnp.sum((counts2 + (_B - 1)) // _B, axis=1).reshape(1)

    tables_k, combine_k = _sc_kernels()
    mapped_slots, slots_pad = tables_k(pos, ppad)
    slots_r = slots_pad.reshape(_G, 1, _B)
    y_pad = _gemm(bexp, nblk, slots_r, hs_bf, w_in, w_out)
    output = combine_k(y_pad, ppad, s0x, s1x)
    return (output, counts, scores, mapped_slots, cum2.reshape(_E))


# speedup vs baseline: 1.1859x; 1.1859x over previous
import functools

import jax
import jax.numpy as jnp
from jax import lax
from jax.experimental import pallas as pl
from jax.experimental.pallas import tpu as pltpu
from jax.experimental.pallas import tpu_sc as plsc

_T = 2048
_D = 1024
_F = 2048
_E = 8
_K = 2
_J = _K * _T
_B = 256
_G = 23
_PAD = _G * _B

_NC = 2
_NS = 16
_NW = _NC * _NS
_JC = _J // _NW
_PC = _PAD // _NW
_TT = _T // _NW


def _route_body(hs_ref, gwt_ref, scores_ref, counts_ref, cum_ref,
                pos_ref, ppad_ref, bexp_ref, scf_ref, hsb_ref):
    hs = hs_ref[...]
    logits = lax.dot_general(
        hs.astype(jnp.bfloat16), gwt_ref[...].astype(jnp.bfloat16),
        (((1,), (0,)), ((), ())), preferred_element_type=jnp.float32)
    m = jnp.max(logits, axis=1, keepdims=True)
    ex = jnp.exp(logits - m)
    probs = ex / jnp.sum(ex, axis=1, keepdims=True)
    lane = lax.broadcasted_iota(jnp.int32, (_T, _E), 1)
    m1 = jnp.max(probs, axis=1, keepdims=True)
    i1 = jnp.min(jnp.where(probs == m1, lane, _E), axis=1, keepdims=True)
    masked = jnp.where(lane == i1, -jnp.inf, probs)
    m2 = jnp.max(masked, axis=1, keepdims=True)
    i2 = jnp.min(jnp.where(masked == m2, lane, _E), axis=1, keepdims=True)
    scores_ref[...] = jnp.concatenate([m1, m2], axis=1)
    scf_ref[...] = jnp.concatenate([m1, m2], axis=0)
    hsb_ref[...] = hs.astype(jnp.bfloat16)

    oh = jnp.concatenate([(lane == i1).astype(jnp.int32),
                          (lane == i2).astype(jnp.int32)], axis=0)
    counts = jnp.sum(oh, axis=0, keepdims=True)
    counts_ref[...] = counts

    def lane_cumsum(v):
        for d in (1, 2, 4):
            v = v + jnp.concatenate(
                [jnp.zeros((1, d), v.dtype), v[:, :_E - d]], axis=1)
        return v

    cum = lane_cumsum(counts)
    cum_ref[...] = cum
    start = cum - counts
    pb = (counts + (_B - 1)) // _B
    startb = lane_cumsum(pb) - pb
    startp = startb * _B

    inc = oh
    d = 1
    while d < _J:
        inc = inc + jnp.concatenate(
            [jnp.zeros((d, _E), jnp.int32), inc[:_J - d, :]], axis=0)
        d *= 2
    rank = jnp.sum(oh * inc, axis=1, keepdims=True) - 1
    pos_ref[...] = rank + jnp.sum(oh * start, axis=1, keepdims=True)
    ppad_ref[...] = rank + jnp.sum(oh * startp, axis=1, keepdims=True)

    eye = (lax.broadcasted_iota(jnp.int32, (_E, _E), 0)
           == lax.broadcasted_iota(jnp.int32, (_E, _E), 1)).astype(jnp.float32)
    startb_sub = lax.dot_general(
        eye, startb.astype(jnp.float32), (((1,), (1,)), ((), ())),
        preferred_element_type=jnp.float32)
    biota = lax.broadcasted_iota(jnp.int32, (_E, _G), 1).astype(jnp.float32)
    cmp = (startb_sub <= biota).astype(jnp.int32)
    bexp_ref[...] = jnp.sum(cmp, axis=0, keepdims=True) - 1


def _route(hidden, gwt):
    return pl.pallas_call(
        _route_body,
        out_shape=(
            jax.ShapeDtypeStruct((_T, _K), jnp.float32),
            jax.ShapeDtypeStruct((1, _E), jnp.int32),
            jax.ShapeDtypeStruct((1, _E), jnp.int32),
            jax.ShapeDtypeStruct((_J, 1), jnp.int32),
            jax.ShapeDtypeStruct((_J, 1), jnp.int32),
            jax.ShapeDtypeStruct((1, _G), jnp.int32),
            jax.ShapeDtypeStruct((_J, 1), jnp.float32),
            jax.ShapeDtypeStruct((_T, _D), jnp.bfloat16),
        ),
    )(hidden, gwt)


def _slots_body(ppad_hbm, scf_hbm, sp_hbm, scp_hbm, ppv, scv, slots, scp):
    wid = lax.axis_index("s") * _NC + lax.axis_index("c")
    slo = wid * _PC
    pltpu.sync_copy(ppad_hbm, ppv)
    pltpu.sync_copy(scf_hbm, scv)
    for i in range(_PC // 16 + 1):
        slots[pl.ds(i * 16, 16)] = jnp.zeros((16,), jnp.int32)
        scp[pl.ds(i * 16, 16)] = jnp.zeros((16,), jnp.float32)

    @plsc.parallel_loop(0, _J // 16, unroll=8)
    def chunk(c):
        pp = ppv[pl.ds(c * 16, 16)]
        sc = scv[pl.ds(c * 16, 16)]
        tok = (c * 16 + lax.iota(jnp.int32, 16)) & (_T - 1)
        m = (pp >= slo) & (pp < slo + _PC)
        plsc.store_scatter(slots, [pp - slo], tok, mask=m)
        plsc.store_scatter(scp, [pp - slo], sc, mask=m)

    pltpu.sync_copy(slots.at[pl.ds(0, _PC)], sp_hbm.at[pl.ds(slo, _PC)])
    pltpu.sync_copy(scp.at[pl.ds(0, _PC)], scp_hbm.at[pl.ds(slo, _PC)])


def _ms_body(pos_hbm, ms_hbm, posv, msbuf):
    wid = lax.axis_index("s") * _NC + lax.axis_index("c")
    mlo = wid * _JC
    pltpu.sync_copy(pos_hbm, posv)

    @plsc.parallel_loop(0, _J // 16, unroll=8)
    def chunk(c):
        pv = posv[pl.ds(c * 16, 16)]
        tok = (c * 16 + lax.iota(jnp.int32, 16)) & (_T - 1)
        plsc.store_scatter(msbuf, [pv - mlo], tok,
                           mask=(pv >= mlo) & (pv < mlo + _JC))

    pltpu.sync_copy(msbuf, ms_hbm.at[pl.ds(mlo, _JC)])


def _gemm_body(be_ref, nb_ref, slots_ref, sp_ref, hid_ref, w1_ref, w2_ref,
               out_ref):
    i = pl.program_id(0)

    @pl.when(i == 0)
    def _():
        out_ref[...] = jnp.zeros((_T, _D), jnp.float32)

    @pl.when(i < nb_ref[0])
    def _():
        s_lane = slots_ref[0].astype(jnp.float32)
        eye = (lax.broadcasted_iota(jnp.int32, (_B, _B), 0)
               == lax.broadcasted_iota(jnp.int32, (_B, _B), 1)
               ).astype(jnp.float32)
        s_col = lax.dot_general(eye, s_lane, (((1,), (1,)), ((), ())),
                                preferred_element_type=jnp.float32)
        t_iota = lax.broadcasted_iota(jnp.int32, (_B, _T), 1).astype(jnp.float32)
        p_blk = (s_col == t_iota).astype(jnp.bfloat16)
        xb = lax.dot_general(p_blk, hid_ref[...], (((1,), (0,)), ((), ())),
                             preferred_element_type=jnp.float32)
        h = lax.dot_general(xb, w1_ref[0],
                            (((1,), (0,)), ((), ())),
                            preferred_element_type=jnp.float32)
        h = h * (1.0 / (1.0 + jnp.exp(-h)))
        y = lax.dot_general(h, w2_ref[0],
                            (((1,), (0,)), ((), ())),
                            preferred_element_type=jnp.float32)
        tt = lax.broadcasted_iota(jnp.int32, (_T, _B), 0).astype(jnp.float32)
        q_blk = jnp.where(tt == s_lane, sp_ref[0], 0.0).astype(jnp.bfloat16)
        out_ref[...] += lax.dot_general(
            q_blk, y.astype(jnp.bfloat16), (((1,), (0,)), ((), ())),
            preferred_element_type=jnp.float32)


def _gemm(bexp, nblk, slots_r, scores_r, hid_bf, w_in, w_out):
    def _clamped(i, be, nb):
        return jnp.minimum(i, nb[0] - 1)

    grid_spec = pltpu.PrefetchScalarGridSpec(
        num_scalar_prefetch=2,
        grid=(_G,),
        in_specs=[
            pl.BlockSpec((1, 1, _B), lambda i, be, nb: (i, 0, 0)),
            pl.BlockSpec((1, 1, _B), lambda i, be, nb: (i, 0, 0)),
            pl.BlockSpec((_T, _D), lambda i, be, nb: (0, 0)),
            pl.BlockSpec((1, _D, _F),
                         lambda i, be, nb: (be[_clamped(i, be, nb)], 0, 0)),
            pl.BlockSpec((1, _F, _D),
                         lambda i, be, nb: (be[_clamped(i, be, nb)], 0, 0)),
        ],
        out_specs=pl.BlockSpec((_T, _D), lambda i, be, nb: (0, 0)),
    )
    return pl.pallas_call(
        _gemm_body,
        grid_spec=grid_spec,
        out_shape=jax.ShapeDtypeStruct((_T, _D), jnp.float32),
    )(bexp, nblk, slots_r, scores_r, hid_bf, w_in, w_out)


@functools.lru_cache(maxsize=1)
def _sc_kernels():
    mesh = plsc.VectorSubcoreMesh(
        core_axis_name="c", subcore_axis_name="s",
        num_cores=_NC, num_subcores=_NS)
    slots_k = pl.kernel(
        _slots_body,
        out_type=(jax.ShapeDtypeStruct((_PAD,), jnp.int32),
                  jax.ShapeDtypeStruct((_PAD,), jnp.float32)),
        mesh=mesh,
        compiler_params=pltpu.CompilerParams(needs_layout_passes=False),
        scratch_types=[pltpu.VMEM((_J,), jnp.int32),
                       pltpu.VMEM((_J,), jnp.float32),
                       pltpu.VMEM((192,), jnp.int32),
                       pltpu.VMEM((192,), jnp.float32)])
    ms_k = pl.kernel(
        _ms_body,
        out_type=jax.ShapeDtypeStruct((_J,), jnp.int32),
        mesh=mesh,
        compiler_params=pltpu.CompilerParams(needs_layout_passes=False),
        scratch_types=[pltpu.VMEM((_J,), jnp.int32),
                       pltpu.VMEM((_JC,), jnp.int32)])
    return slots_k, ms_k


def kernel(hidden_states, gate_w, w_in, w_out):
    (scores, counts2, cum2, pos2, ppad2, bexp2,
     scf2, hs_bf) = _route(hidden_states, gate_w.T)
    counts = counts2.reshape(_E)
    pos = pos2.reshape(_J)
    ppad = ppad2.reshape(_J)
    bexp = bexp2.reshape(_G)
    nblk = jnp.sum((counts2 + (_B - 1)) // _B, axis=1).reshape(1)

    slots_k, ms_k = _sc_kernels()
    slots_pad, scores_pad = slots_k(ppad, scf2.reshape(_J))
    mapped_slots = ms_k(pos)
    slots_r = slots_pad.reshape(_G, 1, _B)
    scores_r = scores_pad.reshape(_G, 1, _B)
    output = _gemm(bexp, nblk, slots_r, scores_r, hs_bf, w_in, w_out)
    return (output, counts, scores, mapped_slots, cum2.reshape(_E))


# speedup vs baseline: 1.1908x; 1.0041x over previous
import functools

import jax
import jax.numpy as jnp
from jax import lax
from jax.experimental import pallas as pl
from jax.experimental.pallas import tpu as pltpu
from jax.experimental.pallas import tpu_sc as plsc

_T = 2048
_D = 1024
_F = 2048
_E = 8
_K = 2
_J = _K * _T
_B = 256
_G = 23
_PAD = _G * _B

_NC = 2
_NS = 16
_NW = _NC * _NS
_JC = _J // _NW
_PC = _PAD // _NW
_TT = _T // _NW


def _route_body(hs_ref, gwt_ref, scores_ref, counts_ref, cum_ref,
                pos_ref, ppad_ref, bexp_ref, scf_ref, hsb_ref):
    hs = hs_ref[...]
    logits = lax.dot_general(
        hs.astype(jnp.bfloat16), gwt_ref[...].astype(jnp.bfloat16),
        (((1,), (0,)), ((), ())), preferred_element_type=jnp.float32)
    m = jnp.max(logits, axis=1, keepdims=True)
    ex = jnp.exp(logits - m)
    probs = ex / jnp.sum(ex, axis=1, keepdims=True)
    lane = lax.broadcasted_iota(jnp.int32, (_T, _E), 1)
    m1 = jnp.max(probs, axis=1, keepdims=True)
    i1 = jnp.min(jnp.where(probs == m1, lane, _E), axis=1, keepdims=True)
    masked = jnp.where(lane == i1, -jnp.inf, probs)
    m2 = jnp.max(masked, axis=1, keepdims=True)
    i2 = jnp.min(jnp.where(masked == m2, lane, _E), axis=1, keepdims=True)
    scores_ref[...] = jnp.concatenate([m1, m2], axis=1)
    scf_ref[...] = jnp.concatenate([m1, m2], axis=0)
    hsb_ref[...] = hs.astype(jnp.bfloat16)

    oh = jnp.concatenate([(lane == i1).astype(jnp.int32),
                          (lane == i2).astype(jnp.int32)], axis=0)
    counts = jnp.sum(oh, axis=0, keepdims=True)
    counts_ref[...] = counts

    def lane_cumsum(v):
        for d in (1, 2, 4):
            v = v + jnp.concatenate(
                [jnp.zeros((1, d), v.dtype), v[:, :_E - d]], axis=1)
        return v

    cum = lane_cumsum(counts)
    cum_ref[...] = cum
    start = cum - counts
    pb = (counts + (_B - 1)) // _B
    startb = lane_cumsum(pb) - pb
    startp = startb * _B

    inc = oh
    d = 1
    while d < _J:
        inc = inc + jnp.concatenate(
            [jnp.zeros((d, _E), jnp.int32), inc[:_J - d, :]], axis=0)
        d *= 2
    rank = jnp.sum(oh * inc, axis=1, keepdims=True) - 1
    pos_ref[...] = rank + jnp.sum(oh * start, axis=1, keepdims=True)
    ppad_ref[...] = rank + jnp.sum(oh * startp, axis=1, keepdims=True)

    eye = (lax.broadcasted_iota(jnp.int32, (_E, _E), 0)
           == lax.broadcasted_iota(jnp.int32, (_E, _E), 1)).astype(jnp.float32)
    startb_sub = lax.dot_general(
        eye, startb.astype(jnp.float32), (((1,), (1,)), ((), ())),
        preferred_element_type=jnp.float32)
    biota = lax.broadcasted_iota(jnp.int32, (_E, _G), 1).astype(jnp.float32)
    cmp = (startb_sub <= biota).astype(jnp.int32)
    bexp_ref[...] = jnp.sum(cmp, axis=0, keepdims=True) - 1


def _route(hidden, gwt):
    return pl.pallas_call(
        _route_body,
        out_shape=(
            jax.ShapeDtypeStruct((_T, _K), jnp.float32),
            jax.ShapeDtypeStruct((1, _E), jnp.int32),
            jax.ShapeDtypeStruct((1, _E), jnp.int32),
            jax.ShapeDtypeStruct((_J, 1), jnp.int32),
            jax.ShapeDtypeStruct((_J, 1), jnp.int32),
            jax.ShapeDtypeStruct((1, _G), jnp.int32),
            jax.ShapeDtypeStruct((_J, 1), jnp.float32),
            jax.ShapeDtypeStruct((_T, _D), jnp.bfloat16),
        ),
    )(hidden, gwt)


def _slots_body(ppad_hbm, scf_hbm, sp_hbm, scp_hbm, ppv, scv, slots, scp):
    wid = lax.axis_index("s") * _NC + lax.axis_index("c")
    slo = wid * _PC
    pltpu.sync_copy(ppad_hbm, ppv)
    pltpu.sync_copy(scf_hbm, scv)
    for i in range(_PC // 16 + 1):
        slots[pl.ds(i * 16, 16)] = jnp.zeros((16,), jnp.int32)
        scp[pl.ds(i * 16, 16)] = jnp.zeros((16,), jnp.float32)

    @plsc.parallel_loop(0, _J // 16, unroll=8)
    def chunk(c):
        pp = ppv[pl.ds(c * 16, 16)]
        sc = scv[pl.ds(c * 16, 16)]
        tok = (c * 16 + lax.iota(jnp.int32, 16)) & (_T - 1)
        m = (pp >= slo) & (pp < slo + _PC)
        plsc.store_scatter(slots, [pp - slo], tok, mask=m)
        plsc.store_scatter(scp, [pp - slo], sc, mask=m)

    pltpu.sync_copy(slots.at[pl.ds(0, _PC)], sp_hbm.at[pl.ds(slo, _PC)])
    pltpu.sync_copy(scp.at[pl.ds(0, _PC)], scp_hbm.at[pl.ds(slo, _PC)])


def _ms_body(pos_hbm, ms_hbm, posv, msbuf):
    wid = lax.axis_index("s") * _NC + lax.axis_index("c")
    mlo = wid * _JC
    pltpu.sync_copy(pos_hbm, posv)

    @plsc.parallel_loop(0, _J // 16, unroll=8)
    def chunk(c):
        pv = posv[pl.ds(c * 16, 16)]
        tok = (c * 16 + lax.iota(jnp.int32, 16)) & (_T - 1)
        plsc.store_scatter(msbuf, [pv - mlo], tok,
                           mask=(pv >= mlo) & (pv < mlo + _JC))

    pltpu.sync_copy(msbuf, ms_hbm.at[pl.ds(mlo, _JC)])


def _gemm_body(be_ref, nb_ref, slots_ref, sp_ref, hid_ref, w1_ref, w2_ref,
               out_ref):
    i = pl.program_id(0)

    @pl.when(i == 0)
    def _():
        out_ref[...] = jnp.zeros((_T, _D), jnp.float32)

    @pl.when(i < nb_ref[0])
    def _():
        s_lane = slots_ref[0].astype(jnp.float32)
        tt = lax.broadcasted_iota(jnp.int32, (_T, _B), 0).astype(jnp.float32)
        pt_blk = (tt == s_lane).astype(jnp.bfloat16)
        xb = lax.dot_general(pt_blk, hid_ref[...], (((0,), (0,)), ((), ())),
                             preferred_element_type=jnp.float32)
        h = lax.dot_general(xb, w1_ref[0],
                            (((1,), (0,)), ((), ())),
                            preferred_element_type=jnp.float32)
        h = h * (1.0 / (1.0 + jnp.exp(-h)))
        y = lax.dot_general(h, w2_ref[0],
                            (((1,), (0,)), ((), ())),
                            preferred_element_type=jnp.float32)
        q_blk = (pt_blk.astype(jnp.float32) * sp_ref[0]).astype(jnp.bfloat16)
        out_ref[...] += lax.dot_general(
            q_blk, y.astype(jnp.bfloat16), (((1,), (0,)), ((), ())),
            preferred_element_type=jnp.float32)


def _gemm(bexp, nblk, slots_r, scores_r, hid_bf, w_in, w_out):
    def _clamped(i, be, nb):
        return jnp.minimum(i, nb[0] - 1)

    grid_spec = pltpu.PrefetchScalarGridSpec(
        num_scalar_prefetch=2,
        grid=(_G,),
        in_specs=[
            pl.BlockSpec((1, 1, _B), lambda i, be, nb: (i, 0, 0)),
            pl.BlockSpec((1, 1, _B), lambda i, be, nb: (i, 0, 0)),
            pl.BlockSpec((_T, _D), lambda i, be, nb: (0, 0)),
            pl.BlockSpec((1, _D, _F),
                         lambda i, be, nb: (be[_clamped(i, be, nb)], 0, 0)),
            pl.BlockSpec((1, _F, _D),
                         lambda i, be, nb: (be[_clamped(i, be, nb)], 0, 0)),
        ],
        out_specs=pl.BlockSpec((_T, _D), lambda i, be, nb: (0, 0)),
    )
    return pl.pallas_call(
        _gemm_body,
        grid_spec=grid_spec,
        out_shape=jax.ShapeDtypeStruct((_T, _D), jnp.float32),
    )(bexp, nblk, slots_r, scores_r, hid_bf, w_in, w_out)


@functools.lru_cache(maxsize=1)
def _sc_kernels():
    mesh = plsc.VectorSubcoreMesh(
        core_axis_name="c", subcore_axis_name="s",
        num_cores=_NC, num_subcores=_NS)
    slots_k = pl.kernel(
        _slots_body,
        out_type=(jax.ShapeDtypeStruct((_PAD,), jnp.int32),
                  jax.ShapeDtypeStruct((_PAD,), jnp.float32)),
        mesh=mesh,
        compiler_params=pltpu.CompilerParams(needs_layout_passes=False),
        scratch_types=[pltpu.VMEM((_J,), jnp.int32),
                       pltpu.VMEM((_J,), jnp.float32),
                       pltpu.VMEM((192,), jnp.int32),
                       pltpu.VMEM((192,), jnp.float32)])
    ms_k = pl.kernel(
        _ms_body,
        out_type=jax.ShapeDtypeStruct((_J,), jnp.int32),
        mesh=mesh,
        compiler_params=pltpu.CompilerParams(needs_layout_passes=False),
        scratch_types=[pltpu.VMEM((_J,), jnp.int32),
                       pltpu.VMEM((_JC,), jnp.int32)])
    return slots_k, ms_k


def kernel(hidden_states, gate_w, w_in, w_out):
    (scores, counts2, cum2, pos2, ppad2, bexp2,
     scf2, hs_bf) = _route(hidden_states, gate_w.T)
    counts = counts2.reshape(_E)
    pos = pos2.reshape(_J)
    ppad = ppad2.reshape(_J)
    bexp = bexp2.reshape(_G)
    nblk = jnp.sum((counts2 + (_B - 1)) // _B, axis=1).reshape(1)

    slots_k, ms_k = _sc_kernels()
    slots_pad, scores_pad = slots_k(ppad, scf2.reshape(_J))
    mapped_slots = ms_k(pos)
    slots_r = slots_pad.reshape(_G, 1, _B)
    scores_r = scores_pad.reshape(_G, 1, _B)
    output = _gemm(bexp, nblk, slots_r, scores_r, hs_bf, w_in, w_out)
    return (output, counts, scores, mapped_slots, cum2.reshape(_E))
